# baseline (speedup 1.0000x reference)
import functools

import jax
import jax.numpy as jnp
from jax import lax
from jax.experimental import pallas as pl
from jax.experimental.pallas import tpu as pltpu
from jax.experimental.pallas import tpu_sc as plsc

DIM = 128
CHUNK = 128

try:
    _info = plsc.get_sparse_core_info()
    NC, NS = _info.num_cores, _info.num_subcores
except Exception:
    NC, NS = 2, 16
NW = NC * NS


def _sc_gather(table, ids3):
    nw, nchunk, chunk = ids3.shape
    n = nw * nchunk * chunk
    per_w = n // NW
    mesh = plsc.VectorSubcoreMesh(core_axis_name="c", subcore_axis_name="s")

    @functools.partial(
        pl.kernel,
        out_type=jax.ShapeDtypeStruct((n, DIM), jnp.float32),
        mesh=mesh,
        scratch_types=[
            pltpu.VMEM((nchunk, CHUNK), jnp.int32),
            pltpu.VMEM((CHUNK, DIM), jnp.float32),
            pltpu.VMEM((CHUNK, DIM), jnp.float32),
            pltpu.SemaphoreType.DMA,
            pltpu.SemaphoreType.DMA,
        ],
    )
    def gather_kernel(table_hbm, ids_hbm, out_hbm, idx_v, buf0, buf1, sem0, sem1):
        wid = lax.axis_index("s") * NC + lax.axis_index("c")
        base = wid * per_w
        pltpu.sync_copy(ids_hbm.at[wid], idx_v)
        bufs = (buf0, buf1)
        sems = (sem0, sem1)

        def start(j, k):
            pltpu.make_async_copy(
                table_hbm.at[idx_v.at[j]], bufs[k], sems[k]
            ).start()

        def finish(j, k):
            pltpu.make_async_copy(
                table_hbm.at[idx_v.at[j]], bufs[k], sems[k]
            ).wait()
            pltpu.sync_copy(bufs[k], out_hbm.at[pl.ds(base + j * CHUNK, CHUNK)])

        start(0, 0)
        start(1, 1)

        def body(i, carry):
            j = 2 * i
            finish(j, 0)

            @pl.when(j + 2 < nchunk)
            def _():
                start(j + 2, 0)

            finish(j + 1, 1)

            @pl.when(j + 3 < nchunk)
            def _():
                start(j + 3, 1)

            return carry

        lax.fori_loop(0, nchunk // 2, body, 0)

    return gather_kernel(table, ids3)


ROWS_PER_BLK = 2048


def _tc_project(g, W, b):
    n = g.shape[0]

    def mm(x_ref, w_ref, b_ref, o_ref):
        o_ref[...] = (
            lax.dot_general(
                x_ref[...],
                w_ref[...],
                (((1,), (1,)), ((), ())),
                preferred_element_type=jnp.float32,
            )
            + b_ref[...]
        )

    return pl.pallas_call(
        mm,
        grid=(n // ROWS_PER_BLK,),
        in_specs=[
            pl.BlockSpec((ROWS_PER_BLK, DIM), lambda i: (i, 0)),
            pl.BlockSpec((DIM, DIM), lambda i: (0, 0)),
            pl.BlockSpec((DIM,), lambda i: (0,)),
        ],
        out_specs=pl.BlockSpec((ROWS_PER_BLK, DIM), lambda i: (i, 0)),
        out_shape=jax.ShapeDtypeStruct((n, DIM), jnp.float32),
    )(g, W, b)


def kernel(news_ids, news_categ, table, W, b):
    B, L = news_ids.shape
    n = B * L
    ids3 = news_ids.reshape(NW, n // (NW * CHUNK), CHUNK).astype(jnp.int32)
    g = _sc_gather(table, ids3)
    out = _tc_project(g, W, b)
    return out.reshape(B, L, DIM)


# speedup vs baseline: 1.3346x; 1.3346x over previous
import functools

import jax
import jax.numpy as jnp
from jax import lax
from jax.experimental import pallas as pl
from jax.experimental.pallas import tpu as pltpu
from jax.experimental.pallas import tpu_sc as plsc

DIM = 128
CHUNK = 128

try:
    _info = plsc.get_sparse_core_info()
    NC, NS = _info.num_cores, _info.num_subcores
except Exception:
    NC, NS = 2, 16
NW = NC * NS


def _sc_gather(table, ids3):
    nw, nchunk, chunk = ids3.shape
    n = nw * nchunk * chunk
    per_w = n // NW
    mesh = plsc.VectorSubcoreMesh(core_axis_name="c", subcore_axis_name="s")

    @functools.partial(
        pl.kernel,
        out_type=jax.ShapeDtypeStruct((n, DIM), jnp.float32),
        mesh=mesh,
        scratch_types=[
            pltpu.VMEM((nchunk, CHUNK), jnp.int32),
            pltpu.VMEM((CHUNK, DIM), jnp.float32),
            pltpu.VMEM((CHUNK, DIM), jnp.float32),
            pltpu.SemaphoreType.DMA,
            pltpu.SemaphoreType.DMA,
        ],
    )
    def gather_kernel(table_hbm, ids_hbm, out_hbm, idx_v, buf0, buf1, sem0, sem1):
        wid = lax.axis_index("s") * NC + lax.axis_index("c")
        base = wid * per_w
        pltpu.sync_copy(ids_hbm.at[wid], idx_v)
        bufs = (buf0, buf1)
        sems = (sem0, sem1)

        def start(j, k):
            pltpu.make_async_copy(
                table_hbm.at[idx_v.at[j]], bufs[k], sems[k]
            ).start()

        def finish(j, k):
            pltpu.make_async_copy(
                table_hbm.at[idx_v.at[j]], bufs[k], sems[k]
            ).wait()
            pltpu.sync_copy(bufs[k], out_hbm.at[pl.ds(base + j * CHUNK, CHUNK)])

        start(0, 0)
        start(1, 1)

        def body(i, carry):
            j = 2 * i
            finish(j, 0)

            @pl.when(j + 2 < nchunk)
            def _():
                start(j + 2, 0)

            finish(j + 1, 1)

            @pl.when(j + 3 < nchunk)
            def _():
                start(j + 3, 1)

            return carry

        lax.fori_loop(0, nchunk // 2, body, 0)

    return gather_kernel(table, ids3)


B_BLK = 32


def _tc_project(g, W, b, B, L):
    rows = B_BLK * L

    def mm(x_ref, w_ref, b_ref, o_ref):
        y = (
            lax.dot_general(
                x_ref[...],
                w_ref[...],
                (((1,), (1,)), ((), ())),
                preferred_element_type=jnp.float32,
            )
            + b_ref[...]
        )
        o_ref[...] = y.reshape(B_BLK, L, DIM)

    return pl.pallas_call(
        mm,
        grid=(B // B_BLK,),
        in_specs=[
            pl.BlockSpec((rows, DIM), lambda i: (i, 0)),
            pl.BlockSpec((DIM, DIM), lambda i: (0, 0)),
            pl.BlockSpec((DIM,), lambda i: (0,)),
        ],
        out_specs=pl.BlockSpec((B_BLK, L, DIM), lambda i: (i, 0, 0)),
        out_shape=jax.ShapeDtypeStruct((B, L, DIM), jnp.float32),
    )(g, W, b)


def kernel(news_ids, news_categ, table, W, b):
    B, L = news_ids.shape
    n = B * L
    ids3 = news_ids.reshape(NW, n // (NW * CHUNK), CHUNK).astype(jnp.int32)
    g = _sc_gather(table, ids3)
    return _tc_project(g, W, b, B, L)


# speedup vs baseline: 2.3336x; 1.7485x over previous
import functools

import jax
import jax.numpy as jnp
from jax import lax
from jax.experimental import pallas as pl
from jax.experimental.pallas import tpu as pltpu
from jax.experimental.pallas import tpu_sc as plsc

DIM = 128
CHUNK = 128

try:
    _info = plsc.get_sparse_core_info()
    NC, NS = _info.num_cores, _info.num_subcores
except Exception:
    NC, NS = 2, 16
NW = NC * NS


def _sc_gather(table, ids3):
    nw, nchunk, chunk = ids3.shape
    n = nw * nchunk * chunk
    per_w = n // NW
    mesh = plsc.VectorSubcoreMesh(core_axis_name="c", subcore_axis_name="s")

    @functools.partial(
        pl.kernel,
        out_type=jax.ShapeDtypeStruct((n, DIM), jnp.float32),
        mesh=mesh,
        scratch_types=[
            pltpu.VMEM((nchunk, CHUNK), jnp.int32),
            pltpu.VMEM((CHUNK, DIM), jnp.float32),
            pltpu.VMEM((CHUNK, DIM), jnp.float32),
            pltpu.SemaphoreType.DMA,
            pltpu.SemaphoreType.DMA,
        ],
    )
    def gather_kernel(table_hbm, ids_hbm, out_hbm, idx_v, buf0, buf1, sem0, sem1):
        wid = lax.axis_index("s") * NC + lax.axis_index("c")
        base = wid * per_w
        pltpu.sync_copy(ids_hbm.at[wid], idx_v)
        bufs = (buf0, buf1)
        sems = (sem0, sem1)

        def start(j, k):
            pltpu.make_async_copy(
                table_hbm.at[idx_v.at[j]], bufs[k], sems[k]
            ).start()

        def finish(j, k):
            pltpu.make_async_copy(
                table_hbm.at[idx_v.at[j]], bufs[k], sems[k]
            ).wait()
            pltpu.sync_copy(bufs[k], out_hbm.at[pl.ds(base + j * CHUNK, CHUNK)])

        start(0, 0)
        start(1, 1)

        def body(i, carry):
            j = 2 * i
            finish(j, 0)

            @pl.when(j + 2 < nchunk)
            def _():
                start(j + 2, 0)

            finish(j + 1, 1)

            @pl.when(j + 3 < nchunk)
            def _():
                start(j + 3, 1)

            return carry

        lax.fori_loop(0, nchunk // 2, body, 0)

    return gather_kernel(table, ids3)


def _tc_project(g, W, b, B, L):
    def mm(x_ref, w_ref, b_ref, o_ref):
        y = (
            lax.dot_general(
                x_ref[...],
                w_ref[...],
                (((1,), (1,)), ((), ())),
                preferred_element_type=jnp.float32,
            )
            + b_ref[...]
        )
        o_ref[...] = y.reshape(1, B, DIM)

    return pl.pallas_call(
        mm,
        grid=(L,),
        in_specs=[
            pl.BlockSpec((B, DIM), lambda i: (i, 0)),
            pl.BlockSpec((DIM, DIM), lambda i: (0, 0)),
            pl.BlockSpec((DIM,), lambda i: (0,)),
        ],
        out_specs=pl.BlockSpec((1, B, DIM), lambda i: (i, 0, 0)),
        out_shape=jax.ShapeDtypeStruct((L, B, DIM), jnp.float32),
    )(g, W, b)


def kernel(news_ids, news_categ, table, W, b):
    B, L = news_ids.shape
    n = B * L
    ids3 = jnp.transpose(news_ids).reshape(NW, n // (NW * CHUNK), CHUNK)
    ids3 = ids3.astype(jnp.int32)
    g = _sc_gather(table, ids3)
    out = _tc_project(g, W, b, B, L)
    return jnp.transpose(out, (1, 0, 2))


# speedup vs baseline: 2.4918x; 1.0678x over previous
import functools

import jax
import jax.numpy as jnp
from jax import lax
from jax.experimental import pallas as pl
from jax.experimental.pallas import tpu as pltpu
from jax.experimental.pallas import tpu_sc as plsc

DIM = 128
CHUNK = 128

try:
    _info = plsc.get_sparse_core_info()
    NC, NS = _info.num_cores, _info.num_subcores
except Exception:
    NC, NS = 2, 16
NW = NC * NS


def _sc_gather(table, ids3):
    nw, nchunk, chunk = ids3.shape
    n = nw * nchunk * chunk
    per_w = n // NW
    mesh = plsc.VectorSubcoreMesh(core_axis_name="c", subcore_axis_name="s")

    @functools.partial(
        pl.kernel,
        out_type=jax.ShapeDtypeStruct((n, DIM), jnp.float32),
        mesh=mesh,
        scratch_types=[
            pltpu.VMEM((nchunk, CHUNK), jnp.int32),
            pltpu.VMEM((CHUNK, DIM), jnp.float32),
            pltpu.VMEM((CHUNK, DIM), jnp.float32),
            pltpu.SemaphoreType.DMA,
            pltpu.SemaphoreType.DMA,
        ],
    )
    def gather_kernel(table_hbm, ids_hbm, out_hbm, idx_v, buf0, buf1, sem0, sem1):
        wid = lax.axis_index("s") * NC + lax.axis_index("c")
        base = wid * per_w
        pltpu.sync_copy(ids_hbm.at[wid], idx_v)
        bufs = (buf0, buf1)
        sems = (sem0, sem1)

        def start(j, k):
            pltpu.make_async_copy(
                table_hbm.at[idx_v.at[j]], bufs[k], sems[k]
            ).start()

        def finish(j, k):
            pltpu.make_async_copy(
                table_hbm.at[idx_v.at[j]], bufs[k], sems[k]
            ).wait()
            pltpu.sync_copy(bufs[k], out_hbm.at[pl.ds(base + j * CHUNK, CHUNK)])

        start(0, 0)
        start(1, 1)

        def body(i, carry):
            j = 2 * i
            finish(j, 0)

            @pl.when(j + 2 < nchunk)
            def _():
                start(j + 2, 0)

            finish(j + 1, 1)

            @pl.when(j + 3 < nchunk)
            def _():
                start(j + 3, 1)

            return carry

        lax.fori_loop(0, nchunk // 2, body, 0)

    return gather_kernel(table, ids3)


L_BLK = 2


def _tc_project(g, W, b, B, L):
    def mm(x_ref, w_ref, b_ref, o_ref):
        y = (
            lax.dot_general(
                x_ref[...].astype(jnp.bfloat16),
                w_ref[...].astype(jnp.bfloat16),
                (((1,), (1,)), ((), ())),
                preferred_element_type=jnp.float32,
            )
            + b_ref[...]
        )
        o_ref[...] = y.reshape(L_BLK, B, DIM)

    return pl.pallas_call(
        mm,
        grid=(L // L_BLK,),
        in_specs=[
            pl.BlockSpec((L_BLK * B, DIM), lambda i: (i, 0)),
            pl.BlockSpec((DIM, DIM), lambda i: (0, 0)),
            pl.BlockSpec((DIM,), lambda i: (0,)),
        ],
        out_specs=pl.BlockSpec((L_BLK, B, DIM), lambda i: (i, 0, 0)),
        out_shape=jax.ShapeDtypeStruct((L, B, DIM), jnp.float32),
    )(g, W, b)


def kernel(news_ids, news_categ, table, W, b):
    B, L = news_ids.shape
    n = B * L
    ids3 = jnp.transpose(news_ids).reshape(NW, n // (NW * CHUNK), CHUNK)
    ids3 = ids3.astype(jnp.int32)
    g = _sc_gather(table, ids3)
    out = _tc_project(g, W, b, B, L)
    return jnp.transpose(out, (1, 0, 2))
